# baseline (speedup 1.0000x reference)
import functools

import jax
import jax.numpy as jnp
from jax import lax
from jax.experimental import pallas as pl
from jax.experimental.pallas import tpu as pltpu
from jax.experimental.pallas import tpu_sc as plsc

N = 10000
NP = 10240
E = 320000
EROWS = E // 128
EMB = 128
HID = 256
G = 256
NUM_CLS = 2
NS = 16

_MESH = plsc.VectorSubcoreMesh(core_axis_name="c", subcore_axis_name="s")

F32 = jnp.float32
I32 = jnp.int32


@functools.partial(
    pl.kernel,
    mesh=_MESH,
    compiler_params=pltpu.CompilerParams(needs_layout_passes=False),
    out_type=[
        jax.ShapeDtypeStruct((NP, 128), F32),
        jax.ShapeDtypeStruct((NP,), F32),
    ],
    scratch_types=[
        pltpu.VMEM((320,), I32),
        pltpu.VMEM((320,), I32),
        pltpu.VMEM((320,), I32),
        pltpu.VMEM((2, 80, 128), F32),
        pltpu.VMEM((2, 80, 128), F32),
        pltpu.VMEM((2, 80, 128), F32),
        pltpu.VMEM((2, 80, 128), F32),
        pltpu.VMEM((80, 128), F32),
        pltpu.VMEM((20000,), I32),
        pltpu.VMEM((8, 128), F32),
        pltpu.VMEM((1, 80), I32),
        pltpu.VMEM((8, 128), F32),
        pltpu.VMEM((1024,), F32),
        pltpu.VMEM_SHARED((80, 128), F32),
        pltpu.SemaphoreType.DMA,
        pltpu.SemaphoreType.DMA,
        pltpu.SemaphoreType.DMA,
        pltpu.SemaphoreType.DMA,
    ],
)
def _emb_deg_kernel(shape_id, colour_id, pos_id, eflat, shape_emb, col_emb,
                    pos_emb, x, recip,
                    sidxb, cidxb, pidxb, bufS2, bufC2, bufP2, xbuf2,
                    hist, dstbuf, zbuf, iotab, dbuf, dbuf1, deg_acc,
                    semG0, semG1, semX0, semX1):
    c = lax.axis_index("c")
    sid = lax.axis_index("s")
    wid = sid * 2 + c
    nb = wid * 320
    semG = [semG0, semG1]
    semX = [semX0, semX1]

    pltpu.sync_copy(shape_id.at[pl.ds(nb, 320)], sidxb)
    pltpu.sync_copy(colour_id.at[pl.ds(nb, 320)], cidxb)
    pltpu.sync_copy(pos_id.at[pl.ds(nb, 320)], pidxb)

    def fire_g(j, sl):
        base = j * 80
        pltpu.async_copy(shape_emb.at[sidxb.at[pl.ds(base, 80)]],
                         bufS2.at[sl], semG[sl])
        pltpu.async_copy(col_emb.at[cidxb.at[pl.ds(base, 80)]],
                         bufC2.at[sl], semG[sl])
        pltpu.async_copy(pos_emb.at[pidxb.at[pl.ds(base, 80)]],
                         bufP2.at[sl], semG[sl])

    def wait_g(sl):
        pltpu.make_async_copy(shape_emb.at[sidxb.at[pl.ds(0, 80)]],
                              bufS2.at[sl], semG[sl]).wait()
        pltpu.make_async_copy(col_emb.at[cidxb.at[pl.ds(0, 80)]],
                              bufC2.at[sl], semG[sl]).wait()
        pltpu.make_async_copy(pos_emb.at[pidxb.at[pl.ds(0, 80)]],
                              bufP2.at[sl], semG[sl]).wait()

    fire_g(0, 0)

    ones16 = jnp.full((16,), 1.0, F32)
    zero16f = jnp.zeros((16,), F32)

    @pl.when(c == 0)
    def _():
        def zhist(r, _):
            for q in range(8):
                hist[r, pl.ds(q * 16, 16)] = zero16f
            return 0

        lax.fori_loop(0, 80, zhist, 0)

        pltpu.sync_copy(eflat.at[pl.ds(E + sid * 20000, 20000)], dstbuf)

        def hrow(i, _):
            dv = dstbuf[pl.ds(i * 16, 16)]
            rv = lax.shift_right_logical(dv, 7)
            cv = lax.bitwise_and(dv, 127)
            plsc.addupdate_scatter(hist, [rv, cv], ones16)
            return 0

        lax.fori_loop(0, 1250, hrow, 0)

    for j in range(4):
        sl = j % 2
        if j < 3:
            fire_g(j + 1, 1 - sl)
        wait_g(sl)
        if j >= 2:
            pltpu.make_async_copy(xbuf2.at[sl], x.at[pl.ds(0, 80)],
                                  semX[sl]).wait()

        def row(r, _):
            for q in range(8):
                qs = pl.ds(q * 16, 16)
                xbuf2[sl, r, qs] = (bufS2[sl, r, qs] + bufC2[sl, r, qs]
                                    + bufP2[sl, r, qs])
            return 0

        lax.fori_loop(0, 80, row, 0)
        pltpu.async_copy(xbuf2.at[sl], x.at[pl.ds(nb + j * 80, 80)], semX[sl])
    for sl in range(2):
        pltpu.make_async_copy(xbuf2.at[sl], x.at[pl.ds(0, 80)],
                              semX[sl]).wait()

    @pl.when(c == 0)
    def _():
        @pl.when(sid < 10)
        def _():
            for r in range(8):
                for q in range(8):
                    zbuf[r, pl.ds(q * 16, 16)] = zero16f
            pltpu.sync_copy(zbuf, deg_acc.at[pl.ds(sid * 8, 8)])

        plsc.subcore_barrier()

        for k in range(5):
            iotab[0, pl.ds(k * 16, 16)] = lax.iota(I32, 16) + k * 16
        pltpu.sync_copy(hist, deg_acc.at[iotab.at[0]], add=True)
        plsc.subcore_barrier()

        @pl.when(sid < 10)
        def _():
            pltpu.sync_copy(deg_acc.at[pl.ds(sid * 8, 8)], dbuf)
            for r in range(8):
                for q in range(8):
                    dbuf1[pl.ds(r * 128 + q * 16, 16)] = (
                        1.0 / jnp.maximum(dbuf[r, pl.ds(q * 16, 16)], 1.0))
            pltpu.sync_copy(dbuf1, recip.at[pl.ds(sid * 1024, 1024)])


def _make_agg_kernel(rt, ex, row_off):
    @functools.partial(
        pl.kernel,
        mesh=_MESH,
        compiler_params=pltpu.CompilerParams(needs_layout_passes=False),
        out_type=[
            jax.ShapeDtypeStruct((NP, 128), F32),
            jax.ShapeDtypeStruct((NP, 128), F32),
        ],
        scratch_types=[
            pltpu.VMEM((2, 32, 128), F32),
            pltpu.VMEM((2, 32), F32),
            pltpu.VMEM((3, 128), I32),
            pltpu.VMEM((3, 128), I32),
            pltpu.VMEM((2, 128, 128), F32),
            pltpu.VMEM_SHARED((NP, 128), F32),
            pltpu.SemaphoreType.DMA,
            pltpu.SemaphoreType.DMA,
            pltpu.SemaphoreType.DMA,
            pltpu.SemaphoreType.DMA,
            pltpu.SemaphoreType.DMA,
        ],
    )
    def agg(x_a, x_b, eflat, recip, out_a, out_b,
            pbuf, rbuf, sidx3, didx3, rows2, acc,
            semi0, semi1, semi2, semg0, semg1):
        c = lax.axis_index("c")
        sid = lax.axis_index("s")
        zero16f = jnp.zeros((16,), F32)
        semi = [semi0, semi1, semi2]
        semg = [semg0, semg1]

        def zrow(r, _):
            for q in range(8):
                pbuf[0, r, pl.ds(q * 16, 16)] = zero16f
            return 0

        lax.fori_loop(0, 32, zrow, 0)
        for k in range(20):
            pltpu.sync_copy(pbuf.at[0], acc.at[pl.ds(sid * 640 + k * 32, 32)])
        plsc.subcore_barrier()

        row0 = c * row_off + sid * rt

        def fire_idx(g, b):
            pltpu.async_copy(eflat.at[pl.ds((row0 + g) * 128, 128)],
                             sidx3.at[b], semi[b])
            pltpu.async_copy(eflat.at[pl.ds(E + (row0 + g) * 128, 128)],
                             didx3.at[b], semi[b])

        def wait_idx(b):
            pltpu.make_async_copy(eflat.at[pl.ds(0, 128)], sidx3.at[b],
                                  semi[b]).wait()
            pltpu.make_async_copy(eflat.at[pl.ds(0, 128)], didx3.at[b],
                                  semi[b]).wait()

        def edge_phase(x_hbm):
            def fire_gather(bi, br):
                pltpu.async_copy(x_hbm.at[sidx3.at[bi]], rows2.at[br],
                                 semg[br])

            def wait_gather(bi, br):
                pltpu.make_async_copy(x_hbm.at[sidx3.at[bi]], rows2.at[br],
                                      semg[br]).wait()

            def scatter(bi, br):
                pltpu.sync_copy(rows2.at[br], acc.at[didx3.at[bi]], add=True)

            fire_idx(0, 0)
            fire_idx(1, 1)

            def step(o, _):
                for b6 in range(6):
                    g = o * 6 + b6
                    bi = b6 % 3
                    br = b6 % 2
                    wait_idx(bi)
                    fire_gather(bi, br)
                    bi1 = (bi + 2) % 3
                    br1 = (br + 1) % 2

                    @pl.when(g >= 1)
                    def _():
                        wait_gather(bi1, br1)
                        scatter(bi1, br1)

                    @pl.when(g + 2 <= rt - 1)
                    def _():
                        fire_idx(g + 2, bi1)
                return 0

            lax.fori_loop(0, rt // 6, step, 0)

            wait_gather((rt - 1) % 3, (rt - 1) % 2)
            scatter((rt - 1) % 3, (rt - 1) % 2)

            @pl.when(sid < ex)
            def _():
                row = c * row_off + NS * rt + sid
                pltpu.sync_copy(eflat.at[pl.ds(row * 128, 128)], sidx3.at[0])
                pltpu.sync_copy(eflat.at[pl.ds(E + row * 128, 128)],
                                didx3.at[0])
                pltpu.async_copy(x_hbm.at[sidx3.at[0]], rows2.at[0],
                                 semg[0]).wait()
                pltpu.sync_copy(rows2.at[0], acc.at[didx3.at[0]], add=True)

        @pl.when(c == 0)
        def _():
            edge_phase(x_a)

        @pl.when(c == 1)
        def _():
            edge_phase(x_b)

        plsc.subcore_barrier()

        def wait_out(sl):
            pltpu.make_async_copy(pbuf.at[sl], out_a.at[pl.ds(0, 32)],
                                  semg[sl]).wait()

        for k in range(20):
            sl = k % 2
            base0 = sid * 640 + k * 32
            if k >= 2:
                wait_out(sl)
            pltpu.sync_copy(acc.at[pl.ds(base0, 32)], pbuf.at[sl])
            pltpu.sync_copy(recip.at[pl.ds(base0, 32)], rbuf.at[sl])

            def srow(b, _):
                rv = rbuf[sl, pl.ds(b * 16, 16)]
                for i in range(16):
                    sv = rv[i]
                    r = b * 16 + i
                    for q in range(8):
                        qs = pl.ds(q * 16, 16)
                        pbuf[sl, r, qs] = pbuf[sl, r, qs] * sv
                return 0

            lax.fori_loop(0, 2, srow, 0)
            base = sid * 640 + k * 32

            @pl.when(c == 0)
            def _():
                pltpu.async_copy(pbuf.at[sl], out_a.at[pl.ds(base, 32)],
                                 semg[sl])

            @pl.when(c == 1)
            def _():
                pltpu.async_copy(pbuf.at[sl], out_b.at[pl.ds(base, 32)],
                                 semg[sl])
        wait_out(0)
        wait_out(1)

    return agg


_agg_l1 = _make_agg_kernel(rt=78, ex=2, row_off=1250)
_agg_l2 = _make_agg_kernel(rt=156, ex=4, row_off=0)


def _dot(a, b):
    return jnp.dot(a, b, preferred_element_type=F32)


BR = 1024


def _linr_body(alo, ahi, wlo, whi, b, o):
    o[...] = _dot(alo[...], wlo[...]) + _dot(ahi[...], whi[...]) + b[...]


def _linr_x_body(a, w, b, o):
    o[...] = _dot(a[...], w[...]) + b[...]


def _linr_x(a, W_r, b):
    return pl.pallas_call(
        _linr_x_body,
        grid=(NP // BR,),
        in_specs=[pl.BlockSpec((BR, 128), lambda i: (i, 0)),
                  pl.BlockSpec((128, 256), lambda i: (0, 0)),
                  pl.BlockSpec((1, 256), lambda i: (0, 0))],
        out_specs=pl.BlockSpec((BR, 256), lambda i: (i, 0)),
        out_shape=jax.ShapeDtypeStruct((NP, HID), F32),
    )(a, W_r, b.reshape(1, HID))


def _linr(a_lo, a_hi, W_r, b):
    row = pl.BlockSpec((BR, 128), lambda i: (i, 0))
    full = lambda sh: pl.BlockSpec(sh, lambda i: (0, 0))
    return pl.pallas_call(
        _linr_body,
        grid=(NP // BR,),
        in_specs=[row, row, full((128, 256)), full((128, 256)),
                  full((1, 256))],
        out_specs=pl.BlockSpec((BR, 256), lambda i: (i, 0)),
        out_shape=jax.ShapeDtypeStruct((NP, HID), F32),
    )(a_lo, a_hi, W_r[:128], W_r[128:], b.reshape(1, HID))


def _layer1_body(p0, p1, wl, z, olo, ohi):
    h = _dot(p0[...] + p1[...], wl[...]) + z[...]
    h = jnp.maximum(h, 0.0)
    olo[...] = h[:, :128]
    ohi[...] = h[:, 128:]


def _layer1(p0, p1, W1_l, z1):
    row = pl.BlockSpec((BR, 128), lambda i: (i, 0))
    full = lambda sh: pl.BlockSpec(sh, lambda i: (0, 0))
    return pl.pallas_call(
        _layer1_body,
        grid=(NP // BR,),
        in_specs=[row, row, full((128, 256)),
                  pl.BlockSpec((BR, 256), lambda i: (i, 0))],
        out_specs=[pl.BlockSpec((BR, 128), lambda i: (i, 0))] * 2,
        out_shape=[jax.ShapeDtypeStruct((NP, 128), F32)] * 2,
    )(p0, p1, W1_l, z1)


def _layer2_body(mlo, mhi, wllo, wlhi, z, bat, wlin, blin, out, pooled, cnts):
    i = pl.program_id(0)
    h2 = _dot(mlo[...], wllo[...]) + _dot(mhi[...], wlhi[...]) + z[...]
    h2 = jnp.maximum(h2, 0.0)

    ids = bat[0]
    gi = lax.broadcasted_iota(I32, (G, BR), 0)
    oh = (gi == ids).astype(F32)

    @pl.when(i == 0)
    def _():
        pooled[...] = jnp.zeros((G, HID), F32)
        cnts[...] = jnp.zeros((G, 8), F32)

    pooled[...] = pooled[...] + _dot(oh, h2)
    cnts[...] = cnts[...] + _dot(oh, jnp.ones((BR, 8), F32))

    @pl.when(i == NP // BR - 1)
    def _():
        cc = jnp.maximum(cnts[:, 0:1], 1.0)
        out[...] = _dot(pooled[...] / cc, wlin[...]) + blin[...]


def _layer2(m_lo, m_hi, W2_l, z2, batch3, W_linp, b_linp):
    row = pl.BlockSpec((BR, 128), lambda i: (i, 0))
    full = lambda sh: pl.BlockSpec(sh, lambda i: (0,) * len(sh))
    return pl.pallas_call(
        _layer2_body,
        grid=(NP // BR,),
        in_specs=[row, row, full((128, 256)), full((128, 256)),
                  pl.BlockSpec((BR, 256), lambda i: (i, 0)),
                  pl.BlockSpec((1, 1, BR), lambda i: (i, 0, 0)),
                  full((256, 128)), full((1, 128))],
        out_specs=full((G, 128)),
        out_shape=jax.ShapeDtypeStruct((G, 128), F32),
        scratch_shapes=[pltpu.VMEM((G, HID), F32), pltpu.VMEM((G, 8), F32)],
    )(m_lo, m_hi, W2_l[:128], W2_l[128:], z2, batch3, W_linp, b_linp)


def kernel(shape_id, colour_id, pos_id, edge_index, batch,
           shape_emb, col_emb, pos_emb,
           W1_l, b1_l, W1_r, W2_l, b2_l, W2_r, W_lin, b_lin):
    pad = NP - N
    zi = jnp.zeros((pad,), I32)
    shape_id_p = jnp.concatenate([shape_id, zi])
    colour_id_p = jnp.concatenate([colour_id, zi])
    pos_id_p = jnp.concatenate([pos_id, zi])
    batch_p = jnp.concatenate([batch, jnp.full((pad,), 2 * G, I32)])
    batch3 = batch_p.reshape(NP // 1024, 1, 1024)

    eflat = edge_index.reshape(2 * E)

    W_linp = jnp.zeros((HID, 128), F32).at[:, :NUM_CLS].set(W_lin)
    b_linp = jnp.zeros((1, 128), F32).at[0, :NUM_CLS].set(b_lin)

    x, recip = _emb_deg_kernel(
        shape_id_p, colour_id_p, pos_id_p, eflat, shape_emb, col_emb, pos_emb)

    p0, p1 = _agg_l1(x, x, eflat, recip)
    z1 = _linr_x(x, W1_r, b1_l)
    h1_lo, h1_hi = _layer1(p0, p1, W1_l, z1)
    m2_lo, m2_hi = _agg_l2(h1_lo, h1_hi, eflat, recip)
    z2 = _linr(h1_lo, h1_hi, W2_r, b2_l)
    out = _layer2(m2_lo, m2_hi, W2_l, z2, batch3, W_linp, b_linp)
    return out[:, :NUM_CLS]


# speedup vs baseline: 1.0434x; 1.0434x over previous
import functools

import jax
import jax.numpy as jnp
from jax import lax
from jax.experimental import pallas as pl
from jax.experimental.pallas import tpu as pltpu
from jax.experimental.pallas import tpu_sc as plsc

N = 10000
NP = 10240
E = 320000
EROWS = E // 128
EMB = 128
HID = 256
G = 256
NUM_CLS = 2
NS = 16

_MESH = plsc.VectorSubcoreMesh(core_axis_name="c", subcore_axis_name="s")

F32 = jnp.float32
I32 = jnp.int32


@functools.partial(
    pl.kernel,
    mesh=_MESH,
    compiler_params=pltpu.CompilerParams(needs_layout_passes=False),
    out_type=[
        jax.ShapeDtypeStruct((NP, 128), F32),
        jax.ShapeDtypeStruct((NP,), F32),
    ],
    scratch_types=[
        pltpu.VMEM((320,), I32),
        pltpu.VMEM((320,), I32),
        pltpu.VMEM((320,), I32),
        pltpu.VMEM((64, 128), F32),
        pltpu.VMEM((16, 128), F32),
        pltpu.VMEM((2, 80, 128), F32),
        pltpu.VMEM((2, 80, 128), F32),
        pltpu.VMEM((80, 128), F32),
        pltpu.VMEM((20000,), I32),
        pltpu.VMEM((8, 128), F32),
        pltpu.VMEM((1, 80), I32),
        pltpu.VMEM((8, 128), F32),
        pltpu.VMEM((1024,), F32),
        pltpu.VMEM_SHARED((80, 128), F32),
        pltpu.SemaphoreType.DMA,
        pltpu.SemaphoreType.DMA,
        pltpu.SemaphoreType.DMA,
        pltpu.SemaphoreType.DMA,
    ],
)
def _emb_deg_kernel(shape_id, colour_id, pos_id, eflat, shape_emb, col_emb,
                    pos_emb, x, recip,
                    sidxb, cidxb, pidxb, shpbuf, colbuf, bufP2, xbuf2,
                    hist, dstbuf, zbuf, iotab, dbuf, dbuf1, deg_acc,
                    semG0, semG1, semX0, semX1):
    c = lax.axis_index("c")
    sid = lax.axis_index("s")
    wid = sid * 2 + c
    nb = wid * 320
    semG = [semG0, semG1]
    semX = [semX0, semX1]

    pltpu.sync_copy(shape_id.at[pl.ds(nb, 320)], sidxb)
    pltpu.sync_copy(colour_id.at[pl.ds(nb, 320)], cidxb)
    pltpu.sync_copy(pos_id.at[pl.ds(nb, 320)], pidxb)
    pltpu.sync_copy(shape_emb, shpbuf)
    pltpu.sync_copy(col_emb, colbuf)

    def fire_g(j, sl):
        base = j * 80
        pltpu.async_copy(pos_emb.at[pidxb.at[pl.ds(base, 80)]],
                         bufP2.at[sl], semG[sl])

    def wait_g(sl):
        pltpu.make_async_copy(pos_emb.at[pidxb.at[pl.ds(0, 80)]],
                              bufP2.at[sl], semG[sl]).wait()

    fire_g(0, 0)

    ones16 = jnp.full((16,), 1.0, F32)
    zero16f = jnp.zeros((16,), F32)

    @pl.when(c == 0)
    def _():
        def zhist(r, _):
            for q in range(8):
                hist[r, pl.ds(q * 16, 16)] = zero16f
            return 0

        lax.fori_loop(0, 80, zhist, 0)

        pltpu.sync_copy(eflat.at[pl.ds(E + sid * 20000, 20000)], dstbuf)

        def hrow(i, _):
            dv = dstbuf[pl.ds(i * 16, 16)]
            rv = lax.shift_right_logical(dv, 7)
            cv = lax.bitwise_and(dv, 127)
            plsc.addupdate_scatter(hist, [rv, cv], ones16)
            return 0

        lax.fori_loop(0, 1250, hrow, 0)

    for j in range(4):
        sl = j % 2
        if j < 3:
            fire_g(j + 1, 1 - sl)
        wait_g(sl)
        if j >= 2:
            pltpu.make_async_copy(xbuf2.at[sl], x.at[pl.ds(0, 80)],
                                  semX[sl]).wait()

        def rowgrp(b, _):
            sv = sidxb[pl.ds(j * 80 + b * 16, 16)]
            cv = cidxb[pl.ds(j * 80 + b * 16, 16)]
            for i in range(16):
                sri = sv[i]
                cri = cv[i]
                r = b * 16 + i
                for q in range(8):
                    qs = pl.ds(q * 16, 16)
                    xbuf2[sl, r, qs] = (shpbuf[sri, qs] + colbuf[cri, qs]
                                        + bufP2[sl, r, qs])
            return 0

        lax.fori_loop(0, 5, rowgrp, 0)
        pltpu.async_copy(xbuf2.at[sl], x.at[pl.ds(nb + j * 80, 80)], semX[sl])
    for sl in range(2):
        pltpu.make_async_copy(xbuf2.at[sl], x.at[pl.ds(0, 80)],
                              semX[sl]).wait()

    @pl.when(c == 0)
    def _():
        @pl.when(sid < 10)
        def _():
            for r in range(8):
                for q in range(8):
                    zbuf[r, pl.ds(q * 16, 16)] = zero16f
            pltpu.sync_copy(zbuf, deg_acc.at[pl.ds(sid * 8, 8)])

        plsc.subcore_barrier()

        for k in range(5):
            iotab[0, pl.ds(k * 16, 16)] = lax.iota(I32, 16) + k * 16
        pltpu.sync_copy(hist, deg_acc.at[iotab.at[0]], add=True)
        plsc.subcore_barrier()

        @pl.when(sid < 10)
        def _():
            pltpu.sync_copy(deg_acc.at[pl.ds(sid * 8, 8)], dbuf)
            for r in range(8):
                for q in range(8):
                    dbuf1[pl.ds(r * 128 + q * 16, 16)] = (
                        1.0 / jnp.maximum(dbuf[r, pl.ds(q * 16, 16)], 1.0))
            pltpu.sync_copy(dbuf1, recip.at[pl.ds(sid * 1024, 1024)])


def _make_agg_kernel(rt, ex, row_off):
    @functools.partial(
        pl.kernel,
        mesh=_MESH,
        compiler_params=pltpu.CompilerParams(needs_layout_passes=False),
        out_type=[
            jax.ShapeDtypeStruct((NP, 128), F32),
            jax.ShapeDtypeStruct((NP, 128), F32),
        ],
        scratch_types=[
            pltpu.VMEM((64, 128), F32),
            pltpu.VMEM((64,), F32),
            pltpu.VMEM((3, 128), I32),
            pltpu.VMEM((3, 128), I32),
            pltpu.VMEM((2, 128, 128), F32),
            pltpu.VMEM_SHARED((NP, 128), F32),
            pltpu.SemaphoreType.DMA,
            pltpu.SemaphoreType.DMA,
            pltpu.SemaphoreType.DMA,
            pltpu.SemaphoreType.DMA,
            pltpu.SemaphoreType.DMA,
        ],
    )
    def agg(x_a, x_b, eflat, recip, out_a, out_b,
            pbuf, rbuf, sidx3, didx3, rows2, acc,
            semi0, semi1, semi2, semg0, semg1):
        c = lax.axis_index("c")
        sid = lax.axis_index("s")
        zero16f = jnp.zeros((16,), F32)
        semi = [semi0, semi1, semi2]
        semg = [semg0, semg1]

        def zrow(r, _):
            for q in range(8):
                pbuf[r, pl.ds(q * 16, 16)] = zero16f
            return 0

        lax.fori_loop(0, 64, zrow, 0)
        for k in range(10):
            pltpu.sync_copy(pbuf, acc.at[pl.ds(sid * 640 + k * 64, 64)])
        plsc.subcore_barrier()

        row0 = c * row_off + sid * rt

        def fire_idx(g, b):
            pltpu.async_copy(eflat.at[pl.ds((row0 + g) * 128, 128)],
                             sidx3.at[b], semi[b])
            pltpu.async_copy(eflat.at[pl.ds(E + (row0 + g) * 128, 128)],
                             didx3.at[b], semi[b])

        def wait_idx(b):
            pltpu.make_async_copy(eflat.at[pl.ds(0, 128)], sidx3.at[b],
                                  semi[b]).wait()
            pltpu.make_async_copy(eflat.at[pl.ds(0, 128)], didx3.at[b],
                                  semi[b]).wait()

        def edge_phase(x_hbm):
            def fire_gather(bi, br):
                pltpu.async_copy(x_hbm.at[sidx3.at[bi]], rows2.at[br],
                                 semg[br])

            def wait_gather(bi, br):
                pltpu.make_async_copy(x_hbm.at[sidx3.at[bi]], rows2.at[br],
                                      semg[br]).wait()

            def scatter(bi, br):
                pltpu.sync_copy(rows2.at[br], acc.at[didx3.at[bi]], add=True)

            fire_idx(0, 0)
            fire_idx(1, 1)

            def step(o, _):
                for b6 in range(6):
                    g = o * 6 + b6
                    bi = b6 % 3
                    br = b6 % 2
                    wait_idx(bi)
                    fire_gather(bi, br)
                    bi1 = (bi + 2) % 3
                    br1 = (br + 1) % 2

                    @pl.when(g >= 1)
                    def _():
                        wait_gather(bi1, br1)
                        scatter(bi1, br1)

                    @pl.when(g + 2 <= rt - 1)
                    def _():
                        fire_idx(g + 2, bi1)
                return 0

            lax.fori_loop(0, rt // 6, step, 0)

            wait_gather((rt - 1) % 3, (rt - 1) % 2)
            scatter((rt - 1) % 3, (rt - 1) % 2)

            @pl.when(sid < ex)
            def _():
                row = c * row_off + NS * rt + sid
                pltpu.sync_copy(eflat.at[pl.ds(row * 128, 128)], sidx3.at[0])
                pltpu.sync_copy(eflat.at[pl.ds(E + row * 128, 128)],
                                didx3.at[0])
                pltpu.async_copy(x_hbm.at[sidx3.at[0]], rows2.at[0],
                                 semg[0]).wait()
                pltpu.sync_copy(rows2.at[0], acc.at[didx3.at[0]], add=True)

        @pl.when(c == 0)
        def _():
            edge_phase(x_a)

        @pl.when(c == 1)
        def _():
            edge_phase(x_b)

        plsc.subcore_barrier()

        for k in range(10):
            base = sid * 640 + k * 64
            pltpu.sync_copy(acc.at[pl.ds(base, 64)], pbuf)
            pltpu.sync_copy(recip.at[pl.ds(base, 64)], rbuf)

            def srow(b, _):
                rv = rbuf[pl.ds(b * 16, 16)]
                for i in range(16):
                    sv = rv[i]
                    r = b * 16 + i
                    for q in range(8):
                        qs = pl.ds(q * 16, 16)
                        pbuf[r, qs] = pbuf[r, qs] * sv
                return 0

            lax.fori_loop(0, 4, srow, 0)

            @pl.when(c == 0)
            def _():
                pltpu.sync_copy(pbuf, out_a.at[pl.ds(base, 64)])

            @pl.when(c == 1)
            def _():
                pltpu.sync_copy(pbuf, out_b.at[pl.ds(base, 64)])

    return agg


_agg_l1 = _make_agg_kernel(rt=78, ex=2, row_off=1250)
_agg_l2 = _make_agg_kernel(rt=156, ex=4, row_off=0)


def _dot(a, b):
    return jnp.dot(a, b, preferred_element_type=F32)


BR = 1024


def _linr_body(alo, ahi, wlo, whi, b, o):
    o[...] = _dot(alo[...], wlo[...]) + _dot(ahi[...], whi[...]) + b[...]


def _linr_x_body(a, w, b, o):
    o[...] = _dot(a[...], w[...]) + b[...]


def _linr_x(a, W_r, b):
    return pl.pallas_call(
        _linr_x_body,
        grid=(NP // BR,),
        in_specs=[pl.BlockSpec((BR, 128), lambda i: (i, 0)),
                  pl.BlockSpec((128, 256), lambda i: (0, 0)),
                  pl.BlockSpec((1, 256), lambda i: (0, 0))],
        out_specs=pl.BlockSpec((BR, 256), lambda i: (i, 0)),
        out_shape=jax.ShapeDtypeStruct((NP, HID), F32),
    )(a, W_r, b.reshape(1, HID))


def _linr(a_lo, a_hi, W_r, b):
    row = pl.BlockSpec((BR, 128), lambda i: (i, 0))
    full = lambda sh: pl.BlockSpec(sh, lambda i: (0, 0))
    return pl.pallas_call(
        _linr_body,
        grid=(NP // BR,),
        in_specs=[row, row, full((128, 256)), full((128, 256)),
                  full((1, 256))],
        out_specs=pl.BlockSpec((BR, 256), lambda i: (i, 0)),
        out_shape=jax.ShapeDtypeStruct((NP, HID), F32),
    )(a_lo, a_hi, W_r[:128], W_r[128:], b.reshape(1, HID))


def _layer1_body(p0, p1, wl, z, olo, ohi):
    h = _dot(p0[...] + p1[...], wl[...]) + z[...]
    h = jnp.maximum(h, 0.0)
    olo[...] = h[:, :128]
    ohi[...] = h[:, 128:]


def _layer1(p0, p1, W1_l, z1):
    row = pl.BlockSpec((BR, 128), lambda i: (i, 0))
    full = lambda sh: pl.BlockSpec(sh, lambda i: (0, 0))
    return pl.pallas_call(
        _layer1_body,
        grid=(NP // BR,),
        in_specs=[row, row, full((128, 256)),
                  pl.BlockSpec((BR, 256), lambda i: (i, 0))],
        out_specs=[pl.BlockSpec((BR, 128), lambda i: (i, 0))] * 2,
        out_shape=[jax.ShapeDtypeStruct((NP, 128), F32)] * 2,
    )(p0, p1, W1_l, z1)


def _layer2_body(mlo, mhi, wllo, wlhi, z, bat, wlin, blin, out, pooled, cnts):
    i = pl.program_id(0)
    h2 = _dot(mlo[...], wllo[...]) + _dot(mhi[...], wlhi[...]) + z[...]
    h2 = jnp.maximum(h2, 0.0)

    ids = bat[0]
    gi = lax.broadcasted_iota(I32, (G, BR), 0)
    oh = (gi == ids).astype(F32)

    @pl.when(i == 0)
    def _():
        pooled[...] = jnp.zeros((G, HID), F32)
        cnts[...] = jnp.zeros((G, 8), F32)

    pooled[...] = pooled[...] + _dot(oh, h2)
    cnts[...] = cnts[...] + _dot(oh, jnp.ones((BR, 8), F32))

    @pl.when(i == NP // BR - 1)
    def _():
        cc = jnp.maximum(cnts[:, 0:1], 1.0)
        out[...] = _dot(pooled[...] / cc, wlin[...]) + blin[...]


def _layer2(m_lo, m_hi, W2_l, z2, batch3, W_linp, b_linp):
    row = pl.BlockSpec((BR, 128), lambda i: (i, 0))
    full = lambda sh: pl.BlockSpec(sh, lambda i: (0,) * len(sh))
    return pl.pallas_call(
        _layer2_body,
        grid=(NP // BR,),
        in_specs=[row, row, full((128, 256)), full((128, 256)),
                  pl.BlockSpec((BR, 256), lambda i: (i, 0)),
                  pl.BlockSpec((1, 1, BR), lambda i: (i, 0, 0)),
                  full((256, 128)), full((1, 128))],
        out_specs=full((G, 128)),
        out_shape=jax.ShapeDtypeStruct((G, 128), F32),
        scratch_shapes=[pltpu.VMEM((G, HID), F32), pltpu.VMEM((G, 8), F32)],
    )(m_lo, m_hi, W2_l[:128], W2_l[128:], z2, batch3, W_linp, b_linp)


def kernel(shape_id, colour_id, pos_id, edge_index, batch,
           shape_emb, col_emb, pos_emb,
           W1_l, b1_l, W1_r, W2_l, b2_l, W2_r, W_lin, b_lin):
    pad = NP - N
    zi = jnp.zeros((pad,), I32)
    shape_id_p = jnp.concatenate([shape_id, zi])
    colour_id_p = jnp.concatenate([colour_id, zi])
    pos_id_p = jnp.concatenate([pos_id, zi])
    batch_p = jnp.concatenate([batch, jnp.full((pad,), 2 * G, I32)])
    batch3 = batch_p.reshape(NP // 1024, 1, 1024)

    eflat = edge_index.reshape(2 * E)

    W_linp = jnp.zeros((HID, 128), F32).at[:, :NUM_CLS].set(W_lin)
    b_linp = jnp.zeros((1, 128), F32).at[0, :NUM_CLS].set(b_lin)

    x, recip = _emb_deg_kernel(
        shape_id_p, colour_id_p, pos_id_p, eflat, shape_emb, col_emb, pos_emb)

    p0, p1 = _agg_l1(x, x, eflat, recip)
    z1 = _linr_x(x, W1_r, b1_l)
    h1_lo, h1_hi = _layer1(p0, p1, W1_l, z1)
    m2_lo, m2_hi = _agg_l2(h1_lo, h1_hi, eflat, recip)
    z2 = _linr(h1_lo, h1_hi, W2_r, b2_l)
    out = _layer2(m2_lo, m2_hi, W2_l, z2, batch3, W_linp, b_linp)
    return out[:, :NUM_CLS]
